# baseline (speedup 1.0000x reference)
import functools

import jax
import jax.numpy as jnp
from jax import lax
from jax.experimental import pallas as pl
from jax.experimental.pallas import tpu as pltpu
from jax.experimental.pallas import tpu_sc as plsc

_N = 10000
_E = 320000
_D = 128
_H = 64
_EPS = 1e-5

_NC = 2
_NS = 16
_NW = _NC * _NS
_EPW = _E // _NW
_CH = 200
_NCHUNK = _EPW // _CH
_NSUPER = _NCHUNK // 2
_SUBS = (104, 96)

_SCH = 128
_NSCH = _EPW // _SCH
_STAIL = _EPW - _NSCH * _SCH
_NSS = _NSCH // 2
_NPAD = 10240
_RPT = _NPAD // _NS
_ZROWS = 128

_L = 16


def _leaky(t):
  return jnp.maximum(t, 0.2 * t)


def _sc_gather_body(src_hbm, dst_hbm, x_hbm, xd_out, g_out, stats_out,
                    idx_s0, idx_d0, idx_s1, idx_d1,
                    rows_s0, rows_d0, rows_s1, rows_d1,
                    acc, sem_g0, sem_g1, sem_w0, sem_w1):
  wid = lax.axis_index("s") * _NC + lax.axis_index("c")
  base0 = wid * _EPW

  zero = jnp.zeros((_L,), jnp.float32)
  for r in range(4):
    for g in range(_D // _L):
      acc[r, pl.ds(g * _L, _L)] = zero

  idx_s = (idx_s0, idx_s1)
  idx_d = (idx_d0, idx_d1)
  rows_s = (rows_s0, rows_s1)
  rows_d = (rows_d0, rows_d1)
  sem_g = (sem_g0, sem_g1)
  sem_w = (sem_w0, sem_w1)

  def fire_gathers(p, c):
    base = base0 + c * _CH
    pltpu.sync_copy(src_hbm.at[pl.ds(base, _CH)], idx_s[p])
    pltpu.sync_copy(dst_hbm.at[pl.ds(base, _CH)], idx_d[p])
    off = 0
    for sub in _SUBS:
      sl = pl.ds(off, sub)
      pltpu.async_copy(x_hbm.at[idx_s[p].at[sl]], rows_s[p].at[sl], sem_g[p])
      pltpu.async_copy(x_hbm.at[idx_d[p].at[sl]], rows_d[p].at[sl], sem_g[p])
      off += sub

  def drain_gathers(p):
    off = 0
    for sub in _SUBS:
      sl = pl.ds(off, sub)
      pltpu.make_async_copy(x_hbm.at[idx_s[p].at[sl]], rows_s[p].at[sl],
                            sem_g[p]).wait()
      pltpu.make_async_copy(x_hbm.at[idx_d[p].at[sl]], rows_d[p].at[sl],
                            sem_g[p]).wait()
      off += sub

  def fire_writes(p, c):
    base = base0 + c * _CH
    pltpu.async_copy(rows_d[p], xd_out.at[pl.ds(base, _CH)], sem_w[p])
    pltpu.async_copy(rows_s[p], g_out.at[pl.ds(base, _CH)], sem_w[p])

  def drain_writes(p):
    pltpu.make_async_copy(rows_d[p], xd_out.at[pl.ds(base0, _CH)],
                          sem_w[p]).wait()
    pltpu.make_async_copy(rows_s[p], g_out.at[pl.ds(base0, _CH)],
                          sem_w[p]).wait()

  _Q = _CH // 4

  def compute(p):
    rs = rows_s[p]
    rd = rows_d[p]
    for g in range(_D // _L):
      sl = pl.ds(g * _L, _L)

      def row_body(j, carry, rs=rs, rd=rd, sl=sl):
        out = []
        for q in range(4):
          ssd, sqd, ssg, sqg = carry[q]
          xd = rd[j + q * _Q, sl]
          xs = rs[j + q * _Q, sl]
          gd = xs - xd
          rs[j + q * _Q, sl] = gd
          out.append((ssd + xd, sqd + xd * xd, ssg + gd, sqg + gd * gd))
        return tuple(out)

      parts = plsc.parallel_loop(
          0, _Q, carry=((zero, zero, zero, zero),) * 4, unroll=2)(row_body)
      ssd = (parts[0][0] + parts[1][0]) + (parts[2][0] + parts[3][0])
      sqd = (parts[0][1] + parts[1][1]) + (parts[2][1] + parts[3][1])
      ssg = (parts[0][2] + parts[1][2]) + (parts[2][2] + parts[3][2])
      sqg = (parts[0][3] + parts[1][3]) + (parts[2][3] + parts[3][3])
      acc[0, sl] = acc[0, sl] + ssd
      acc[1, sl] = acc[1, sl] + sqd
      acc[2, sl] = acc[2, sl] + ssg
      acc[3, sl] = acc[3, sl] + sqg

  fire_gathers(0, 0)

  def super_body(t, _):
    @pl.when(t > 0)
    def _():
      drain_writes(1)
    fire_gathers(1, 2 * t + 1)

    drain_gathers(0)
    compute(0)
    fire_writes(0, 2 * t)

    @pl.when(t < _NSUPER - 1)
    def _():
      drain_writes(0)
      fire_gathers(0, 2 * t + 2)

    drain_gathers(1)
    compute(1)
    fire_writes(1, 2 * t + 1)
    return 0

  lax.fori_loop(0, _NSUPER, super_body, 0)
  drain_writes(0)
  drain_writes(1)
  pltpu.sync_copy(acc, stats_out.at[wid])


_sc_gather = functools.partial(
    pl.kernel,
    out_type=[
        jax.ShapeDtypeStruct((_E, _D), jnp.float32),
        jax.ShapeDtypeStruct((_E, _D), jnp.float32),
        jax.ShapeDtypeStruct((_NW, 4, _D), jnp.float32),
    ],
    mesh=plsc.VectorSubcoreMesh(core_axis_name="c", subcore_axis_name="s"),
    scratch_types=[
        pltpu.VMEM((_CH,), jnp.int32),
        pltpu.VMEM((_CH,), jnp.int32),
        pltpu.VMEM((_CH,), jnp.int32),
        pltpu.VMEM((_CH,), jnp.int32),
        pltpu.VMEM((_CH, _D), jnp.float32),
        pltpu.VMEM((_CH, _D), jnp.float32),
        pltpu.VMEM((_CH, _D), jnp.float32),
        pltpu.VMEM((_CH, _D), jnp.float32),
        pltpu.VMEM((4, _D), jnp.float32),
        pltpu.SemaphoreType.DMA,
        pltpu.SemaphoreType.DMA,
        pltpu.SemaphoreType.DMA,
        pltpu.SemaphoreType.DMA,
    ],
)(_sc_gather_body)


_BB = 2560
_NSTEP = _E // _BB


def _tc_mlp1_body(partials, g1, b1, w1at, w1bt, g2, b2,
                  xd_blk, g_blk, z_out, ab2_out, acc_s, acc_q):
  i = pl.program_id(0)
  sums = jnp.sum(partials[...], axis=0)
  mean_a = sums[0:1, :] / _E
  var_a = sums[1:2, :] / _E - mean_a * mean_a
  a1a = g1[:, 0:_D] * lax.rsqrt(var_a + _EPS)
  b1a = b1[:, 0:_D] - mean_a * a1a
  mean_b = sums[2:3, :] / _E
  var_b = sums[3:4, :] / _E - mean_b * mean_b
  a1b = g1[:, _D:] * lax.rsqrt(var_b + _EPS)
  b1b = b1[:, _D:] - mean_b * a1b

  ya = _leaky(xd_blk[...] * a1a + b1a)
  yb = _leaky(g_blk[...] * a1b + b1b)
  z = (jnp.dot(ya, w1at[...], preferred_element_type=jnp.float32)
       + jnp.dot(yb, w1bt[...], preferred_element_type=jnp.float32))
  z_out[...] = z

  s_blk = jnp.sum(z, axis=0, keepdims=True)
  q_blk = jnp.sum(z * z, axis=0, keepdims=True)

  @pl.when(i == 0)
  def _():
    acc_s[...] = s_blk
    acc_q[...] = q_blk

  @pl.when(i > 0)
  def _():
    acc_s[...] = acc_s[...] + s_blk
    acc_q[...] = acc_q[...] + q_blk

  @pl.when(i == _NSTEP - 1)
  def _():
    mean2 = acc_s[...] / _E
    var2 = acc_q[...] / _E - mean2 * mean2
    a2 = g2[...] * lax.rsqrt(var2 + _EPS)
    b2v = b2[...] - mean2 * a2
    ab2_out[0:1, :] = a2
    ab2_out[1:2, :] = b2v


_tc_mlp1 = pl.pallas_call(
    _tc_mlp1_body,
    grid=(_NSTEP,),
    in_specs=[
        pl.BlockSpec((_NW, 4, _D), lambda i: (0, 0, 0)),
        pl.BlockSpec((1, 2 * _D), lambda i: (0, 0)),
        pl.BlockSpec((1, 2 * _D), lambda i: (0, 0)),
        pl.BlockSpec((_D, _H), lambda i: (0, 0)),
        pl.BlockSpec((_D, _H), lambda i: (0, 0)),
        pl.BlockSpec((1, _H), lambda i: (0, 0)),
        pl.BlockSpec((1, _H), lambda i: (0, 0)),
        pl.BlockSpec((_BB, _D), lambda i: (i, 0)),
        pl.BlockSpec((_BB, _D), lambda i: (i, 0)),
    ],
    out_specs=[
        pl.BlockSpec((_BB, _H), lambda i: (i, 0)),
        pl.BlockSpec((2, _H), lambda i: (0, 0)),
    ],
    out_shape=[
        jax.ShapeDtypeStruct((_E, _H), jnp.float32),
        jax.ShapeDtypeStruct((2, _H), jnp.float32),
    ],
    scratch_shapes=[
        pltpu.VMEM((1, _H), jnp.float32),
        pltpu.VMEM((1, _H), jnp.float32),
    ],
)


def _tc_bn2_body(ab2, z_blk, ew_blk, v_out):
  a2 = ab2[0:1, :]
  b2v = ab2[1:2, :]
  v_out[...] = ew_blk[...] * _leaky(z_blk[...] * a2 + b2v)


_tc_bn2 = pl.pallas_call(
    _tc_bn2_body,
    grid=(_NSTEP,),
    in_specs=[
        pl.BlockSpec((2, _H), lambda i: (0, 0)),
        pl.BlockSpec((_BB, _H), lambda i: (i, 0)),
        pl.BlockSpec((_BB, _H), lambda i: (i, 0)),
    ],
    out_specs=pl.BlockSpec((_BB, _H), lambda i: (i, 0)),
    out_shape=jax.ShapeDtypeStruct((_E, _H), jnp.float32),
)


def _sc_scatter_body(dst_hbm, v_hbm, out_hbm,
                     idx0, idx1, idxt, vr0, vr1, vrt,
                     shared_acc, sem_l0, sem_l1):
  cid = lax.axis_index("c")
  sid = lax.axis_index("s")
  wid = sid * _NC + cid
  base0 = wid * _EPW

  zero = jnp.zeros((_L,), jnp.float32)

  def zrow(j, _):
    for g in range(_H // _L):
      vr0[j, pl.ds(g * _L, _L)] = zero
    return 0

  lax.fori_loop(0, _ZROWS, zrow, 0)
  for k in range(_RPT // _ZROWS):
    pltpu.sync_copy(vr0, shared_acc.at[pl.ds(sid * _RPT + k * _ZROWS, _ZROWS)])
  plsc.subcore_barrier()

  idx = (idx0, idx1)
  vr = (vr0, vr1)
  sem_l = (sem_l0, sem_l1)

  def fire_load(p, c):
    base = base0 + c * _SCH
    pltpu.async_copy(dst_hbm.at[pl.ds(base, _SCH)], idx[p].at[0], sem_l[p])
    pltpu.async_copy(v_hbm.at[pl.ds(base, _SCH)], vr[p], sem_l[p])

  def drain_load(p):
    pltpu.make_async_copy(dst_hbm.at[pl.ds(base0, _SCH)], idx[p].at[0],
                          sem_l[p]).wait()
    pltpu.make_async_copy(v_hbm.at[pl.ds(base0, _SCH)], vr[p],
                          sem_l[p]).wait()

  def scatter(p):
    pltpu.sync_copy(vr[p], shared_acc.at[idx[p].at[0]], add=True)
    plsc.subcore_barrier()

  fire_load(0, 0)

  def super_body(t, _):
    fire_load(1, 2 * t + 1)
    drain_load(0)
    scatter(0)

    @pl.when(t < _NSS - 1)
    def _():
      fire_load(0, 2 * t + 2)

    drain_load(1)
    scatter(1)
    return 0

  lax.fori_loop(0, _NSS, super_body, 0)

  tbase = base0 + _NSCH * _SCH
  pltpu.sync_copy(dst_hbm.at[pl.ds(tbase, _STAIL)], idxt.at[0])
  pltpu.sync_copy(v_hbm.at[pl.ds(tbase, _STAIL)], vrt)
  pltpu.sync_copy(vrt, shared_acc.at[idxt.at[0]], add=True)
  plsc.subcore_barrier()

  pltpu.sync_copy(shared_acc.at[pl.ds(sid * _RPT, _RPT)],
                  out_hbm.at[cid, pl.ds(sid * _RPT, _RPT)])


_sc_scatter = functools.partial(
    pl.kernel,
    out_type=jax.ShapeDtypeStruct((_NC, _NPAD, _H), jnp.float32),
    mesh=plsc.VectorSubcoreMesh(core_axis_name="c", subcore_axis_name="s"),
    scratch_types=[
        pltpu.VMEM((1, _SCH), jnp.int32),
        pltpu.VMEM((1, _SCH), jnp.int32),
        pltpu.VMEM((1, _STAIL), jnp.int32),
        pltpu.VMEM((_SCH, _H), jnp.float32),
        pltpu.VMEM((_SCH, _H), jnp.float32),
        pltpu.VMEM((_STAIL, _H), jnp.float32),
        pltpu.VMEM_SHARED((_NPAD, _H), jnp.float32),
        pltpu.SemaphoreType.DMA,
        pltpu.SemaphoreType.DMA,
    ],
    compiler_params=pltpu.CompilerParams(use_tc_tiling_on_sc=False),
)(_sc_scatter_body)


def _tc_out_body(p, w2t, out):
  out[...] = jnp.dot(p[0] + p[1], w2t[...], preferred_element_type=jnp.float32)


_tc_out = pl.pallas_call(
    _tc_out_body,
    out_shape=jax.ShapeDtypeStruct((_N, _H), jnp.float32),
)


def kernel(x, edge_index, edge_weight, bn1_gamma, bn1_beta, W1,
           bn2_gamma, bn2_beta, W2):
  src = edge_index[0]
  dst = edge_index[1]
  xd, g, partials = _sc_gather(src, dst, x)
  z, ab2 = _tc_mlp1(
      partials,
      bn1_gamma.reshape(1, -1), bn1_beta.reshape(1, -1),
      W1[:, :_D].T, W1[:, _D:].T,
      bn2_gamma.reshape(1, -1), bn2_beta.reshape(1, -1),
      xd, g)
  ewb = jnp.broadcast_to(edge_weight[:, None], (_E, _H))
  v = _tc_bn2(ab2, z, ewb)
  p = _sc_scatter(dst, v)
  return _tc_out(p[:, :_N], W2.T)


# speedup vs baseline: 1.2506x; 1.2506x over previous
import functools

import jax
import jax.numpy as jnp
from jax import lax
from jax.experimental import pallas as pl
from jax.experimental.pallas import tpu as pltpu
from jax.experimental.pallas import tpu_sc as plsc

_N = 10000
_E = 320000
_D = 128
_H = 64
_EPS = 1e-5

_NC = 2
_NS = 16
_NW = _NC * _NS
_EPW = _E // _NW
_CH = 200
_NCHUNK = _EPW // _CH
_NSUPER = _NCHUNK // 2
_SUBS = (104, 96)

_SCH = 128
_NSCH = _EPW // _SCH
_STAIL = _EPW - _NSCH * _SCH
_NSS = _NSCH // 2
_NPAD = 10240
_RPT = _NPAD // _NS
_ZROWS = 128

_L = 16


def _leaky(t):
  return jnp.maximum(t, 0.2 * t)


def _sc_gather_body(src_hbm, dst_hbm, x_hbm, xd_out, g_out, stats_out,
                    idx_s0, idx_d0, idx_s1, idx_d1,
                    rows_s0, rows_d0, rows_s1, rows_d1,
                    acc, sem_g0, sem_g1, sem_w0, sem_w1):
  wid = lax.axis_index("s") * _NC + lax.axis_index("c")
  base0 = wid * _EPW

  zero = jnp.zeros((_L,), jnp.float32)
  for r in range(4):
    for g in range(_D // _L):
      acc[r, pl.ds(g * _L, _L)] = zero

  idx_s = (idx_s0, idx_s1)
  idx_d = (idx_d0, idx_d1)
  rows_s = (rows_s0, rows_s1)
  rows_d = (rows_d0, rows_d1)
  sem_g = (sem_g0, sem_g1)
  sem_w = (sem_w0, sem_w1)

  def fire_gathers(p, c):
    base = base0 + c * _CH
    pltpu.sync_copy(src_hbm.at[pl.ds(base, _CH)], idx_s[p])
    pltpu.sync_copy(dst_hbm.at[pl.ds(base, _CH)], idx_d[p])
    off = 0
    for sub in _SUBS:
      sl = pl.ds(off, sub)
      pltpu.async_copy(x_hbm.at[idx_s[p].at[sl]], rows_s[p].at[sl], sem_g[p])
      pltpu.async_copy(x_hbm.at[idx_d[p].at[sl]], rows_d[p].at[sl], sem_g[p])
      off += sub

  def drain_gathers(p):
    off = 0
    for sub in _SUBS:
      sl = pl.ds(off, sub)
      pltpu.make_async_copy(x_hbm.at[idx_s[p].at[sl]], rows_s[p].at[sl],
                            sem_g[p]).wait()
      pltpu.make_async_copy(x_hbm.at[idx_d[p].at[sl]], rows_d[p].at[sl],
                            sem_g[p]).wait()
      off += sub

  def fire_writes(p, c):
    base = base0 + c * _CH
    pltpu.async_copy(rows_d[p], xd_out.at[pl.ds(base, _CH)], sem_w[p])
    pltpu.async_copy(rows_s[p], g_out.at[pl.ds(base, _CH)], sem_w[p])

  def drain_writes(p):
    pltpu.make_async_copy(rows_d[p], xd_out.at[pl.ds(base0, _CH)],
                          sem_w[p]).wait()
    pltpu.make_async_copy(rows_s[p], g_out.at[pl.ds(base0, _CH)],
                          sem_w[p]).wait()

  _Q = _CH // 4

  def compute(p):
    rs = rows_s[p]
    rd = rows_d[p]
    for g in range(_D // _L):
      sl = pl.ds(g * _L, _L)

      def row_body(j, carry, rs=rs, rd=rd, sl=sl):
        out = []
        for q in range(4):
          ssd, sqd, ssg, sqg = carry[q]
          xd = rd[j + q * _Q, sl]
          xs = rs[j + q * _Q, sl]
          gd = xs - xd
          rs[j + q * _Q, sl] = gd
          out.append((ssd + xd, sqd + xd * xd, ssg + gd, sqg + gd * gd))
        return tuple(out)

      parts = plsc.parallel_loop(
          0, _Q, carry=((zero, zero, zero, zero),) * 4, unroll=2)(row_body)
      ssd = (parts[0][0] + parts[1][0]) + (parts[2][0] + parts[3][0])
      sqd = (parts[0][1] + parts[1][1]) + (parts[2][1] + parts[3][1])
      ssg = (parts[0][2] + parts[1][2]) + (parts[2][2] + parts[3][2])
      sqg = (parts[0][3] + parts[1][3]) + (parts[2][3] + parts[3][3])
      acc[0, sl] = acc[0, sl] + ssd
      acc[1, sl] = acc[1, sl] + sqd
      acc[2, sl] = acc[2, sl] + ssg
      acc[3, sl] = acc[3, sl] + sqg

  fire_gathers(0, 0)

  def super_body(t, _):
    @pl.when(t > 0)
    def _():
      drain_writes(1)
    fire_gathers(1, 2 * t + 1)

    drain_gathers(0)
    compute(0)
    fire_writes(0, 2 * t)

    @pl.when(t < _NSUPER - 1)
    def _():
      drain_writes(0)
      fire_gathers(0, 2 * t + 2)

    drain_gathers(1)
    compute(1)
    fire_writes(1, 2 * t + 1)
    return 0

  lax.fori_loop(0, _NSUPER, super_body, 0)
  drain_writes(0)
  drain_writes(1)
  pltpu.sync_copy(acc, stats_out.at[wid])


_sc_gather = functools.partial(
    pl.kernel,
    out_type=[
        jax.ShapeDtypeStruct((_E, _D), jnp.float32),
        jax.ShapeDtypeStruct((_E, _D), jnp.float32),
        jax.ShapeDtypeStruct((_NW, 4, _D), jnp.float32),
    ],
    mesh=plsc.VectorSubcoreMesh(core_axis_name="c", subcore_axis_name="s"),
    scratch_types=[
        pltpu.VMEM((_CH,), jnp.int32),
        pltpu.VMEM((_CH,), jnp.int32),
        pltpu.VMEM((_CH,), jnp.int32),
        pltpu.VMEM((_CH,), jnp.int32),
        pltpu.VMEM((_CH, _D), jnp.float32),
        pltpu.VMEM((_CH, _D), jnp.float32),
        pltpu.VMEM((_CH, _D), jnp.float32),
        pltpu.VMEM((_CH, _D), jnp.float32),
        pltpu.VMEM((4, _D), jnp.float32),
        pltpu.SemaphoreType.DMA,
        pltpu.SemaphoreType.DMA,
        pltpu.SemaphoreType.DMA,
        pltpu.SemaphoreType.DMA,
    ],
)(_sc_gather_body)


_BB = 2560
_NSTEP = _E // _BB


def _tc_mlp1_body(partials, g1, b1, w1at, w1bt, g2, b2,
                  xd_blk, g_blk, z_out, ab2_out, acc_s, acc_q):
  i = pl.program_id(0)
  sums = jnp.sum(partials[...], axis=0)
  mean_a = sums[0:1, :] / _E
  var_a = sums[1:2, :] / _E - mean_a * mean_a
  a1a = g1[:, 0:_D] * lax.rsqrt(var_a + _EPS)
  b1a = b1[:, 0:_D] - mean_a * a1a
  mean_b = sums[2:3, :] / _E
  var_b = sums[3:4, :] / _E - mean_b * mean_b
  a1b = g1[:, _D:] * lax.rsqrt(var_b + _EPS)
  b1b = b1[:, _D:] - mean_b * a1b

  ya = _leaky(xd_blk[...] * a1a + b1a)
  yb = _leaky(g_blk[...] * a1b + b1b)
  z = (jnp.dot(ya, w1at[...], preferred_element_type=jnp.float32)
       + jnp.dot(yb, w1bt[...], preferred_element_type=jnp.float32))
  z_out[...] = z

  s_blk = jnp.sum(z, axis=0, keepdims=True)
  q_blk = jnp.sum(z * z, axis=0, keepdims=True)

  @pl.when(i == 0)
  def _():
    acc_s[...] = s_blk
    acc_q[...] = q_blk

  @pl.when(i > 0)
  def _():
    acc_s[...] = acc_s[...] + s_blk
    acc_q[...] = acc_q[...] + q_blk

  @pl.when(i == _NSTEP - 1)
  def _():
    mean2 = acc_s[...] / _E
    var2 = acc_q[...] / _E - mean2 * mean2
    a2 = g2[...] * lax.rsqrt(var2 + _EPS)
    b2v = b2[...] - mean2 * a2
    ab2_out[0:1, :] = a2
    ab2_out[1:2, :] = b2v


_tc_mlp1 = pl.pallas_call(
    _tc_mlp1_body,
    grid=(_NSTEP,),
    in_specs=[
        pl.BlockSpec((_NW, 4, _D), lambda i: (0, 0, 0)),
        pl.BlockSpec((1, 2 * _D), lambda i: (0, 0)),
        pl.BlockSpec((1, 2 * _D), lambda i: (0, 0)),
        pl.BlockSpec((_D, _H), lambda i: (0, 0)),
        pl.BlockSpec((_D, _H), lambda i: (0, 0)),
        pl.BlockSpec((1, _H), lambda i: (0, 0)),
        pl.BlockSpec((1, _H), lambda i: (0, 0)),
        pl.BlockSpec((_BB, _D), lambda i: (i, 0)),
        pl.BlockSpec((_BB, _D), lambda i: (i, 0)),
    ],
    out_specs=[
        pl.BlockSpec((_BB, _H), lambda i: (i, 0)),
        pl.BlockSpec((2, _H), lambda i: (0, 0)),
    ],
    out_shape=[
        jax.ShapeDtypeStruct((_E, _H), jnp.float32),
        jax.ShapeDtypeStruct((2, _H), jnp.float32),
    ],
    scratch_shapes=[
        pltpu.VMEM((1, _H), jnp.float32),
        pltpu.VMEM((1, _H), jnp.float32),
    ],
)


def _sc_scatter_body(dst_hbm, z_hbm, ew_hbm, ab2_hbm, out_hbm,
                     idx0, idx1, idxt, vr0, vr1, vrt, zb0, zb1, zbt,
                     ew0, ew1, ewt, ab2_v,
                     shared_acc, sem_l0, sem_l1):
  cid = lax.axis_index("c")
  sid = lax.axis_index("s")
  wid = sid * _NC + cid
  base0 = wid * _EPW

  zero = jnp.zeros((_L,), jnp.float32)

  pltpu.sync_copy(ab2_hbm, ab2_v)

  def zrow(j, _):
    for g in range(_H // _L):
      vr0[j, pl.ds(g * _L, _L)] = zero
    return 0

  lax.fori_loop(0, _ZROWS, zrow, 0)
  for k in range(_RPT // _ZROWS):
    pltpu.sync_copy(vr0, shared_acc.at[pl.ds(sid * _RPT + k * _ZROWS, _ZROWS)])
  plsc.subcore_barrier()

  a2g = [ab2_v[0, pl.ds(k * _L, _L)] for k in range(_H // _L)]
  b2g = [ab2_v[1, pl.ds(k * _L, _L)] for k in range(_H // _L)]

  idx = (idx0, idx1)
  vr = (vr0, vr1)
  zb = (zb0, zb1)
  ew = (ew0, ew1)
  sem_l = (sem_l0, sem_l1)

  def fire_load(p, c):
    base = base0 + c * _SCH
    pltpu.async_copy(dst_hbm.at[pl.ds(base, _SCH)], idx[p].at[0], sem_l[p])
    pltpu.async_copy(z_hbm.at[pl.ds(base, _SCH)], zb[p], sem_l[p])
    pltpu.async_copy(ew_hbm.at[pl.ds(base, _SCH)], ew[p], sem_l[p])

  def drain_load(p):
    pltpu.make_async_copy(dst_hbm.at[pl.ds(base0, _SCH)], idx[p].at[0],
                          sem_l[p]).wait()
    pltpu.make_async_copy(z_hbm.at[pl.ds(base0, _SCH)], zb[p],
                          sem_l[p]).wait()
    pltpu.make_async_copy(ew_hbm.at[pl.ds(base0, _SCH)], ew[p],
                          sem_l[p]).wait()

  def compute(p):
    buf = vr[p]
    src = zb[p]
    ewb = ew[p]

    def grp_body(gg):
      ewv = ewb[pl.ds(gg * _L, _L)]
      for i in range(_L):
        j = gg * _L + i
        ewj = ewv[i]
        for k in range(_H // _L):
          sl = pl.ds(k * _L, _L)
          t = src[j, sl] * a2g[k] + b2g[k]
          buf[j, sl] = jnp.maximum(t, 0.2 * t) * ewj

    plsc.parallel_loop(0, _SCH // _L, unroll=1)(grp_body)

  def scatter(p):
    pltpu.sync_copy(vr[p], shared_acc.at[idx[p].at[0]], add=True)
    plsc.subcore_barrier()

  fire_load(0, 0)

  def super_body(t, _):
    fire_load(1, 2 * t + 1)
    drain_load(0)
    compute(0)
    scatter(0)

    @pl.when(t < _NSS - 1)
    def _():
      fire_load(0, 2 * t + 2)

    drain_load(1)
    compute(1)
    scatter(1)
    return 0

  lax.fori_loop(0, _NSS, super_body, 0)

  tbase = base0 + _NSCH * _SCH
  pltpu.sync_copy(dst_hbm.at[pl.ds(tbase, _STAIL)], idxt.at[0])
  pltpu.sync_copy(z_hbm.at[pl.ds(tbase, _STAIL)], zbt)
  pltpu.sync_copy(ew_hbm.at[pl.ds(tbase, _STAIL)], ewt)

  ewtv = ewt[pl.ds(0, _L)]
  for i in range(_STAIL):
    ewj = ewtv[i]
    for k in range(_H // _L):
      sl = pl.ds(k * _L, _L)
      t = zbt[i, sl] * a2g[k] + b2g[k]
      vrt[i, sl] = jnp.maximum(t, 0.2 * t) * ewj
  pltpu.sync_copy(vrt, shared_acc.at[idxt.at[0]], add=True)
  plsc.subcore_barrier()

  pltpu.sync_copy(shared_acc.at[pl.ds(sid * _RPT, _RPT)],
                  out_hbm.at[cid, pl.ds(sid * _RPT, _RPT)])


_sc_scatter = functools.partial(
    pl.kernel,
    out_type=jax.ShapeDtypeStruct((_NC, _NPAD, _H), jnp.float32),
    mesh=plsc.VectorSubcoreMesh(core_axis_name="c", subcore_axis_name="s"),
    scratch_types=[
        pltpu.VMEM((1, _SCH), jnp.int32),
        pltpu.VMEM((1, _SCH), jnp.int32),
        pltpu.VMEM((1, _STAIL), jnp.int32),
        pltpu.VMEM((_SCH, _H), jnp.float32),
        pltpu.VMEM((_SCH, _H), jnp.float32),
        pltpu.VMEM((_STAIL, _H), jnp.float32),
        pltpu.VMEM((_SCH, _H), jnp.float32),
        pltpu.VMEM((_SCH, _H), jnp.float32),
        pltpu.VMEM((_STAIL, _H), jnp.float32),
        pltpu.VMEM((_SCH,), jnp.float32),
        pltpu.VMEM((_SCH,), jnp.float32),
        pltpu.VMEM((_STAIL,), jnp.float32),
        pltpu.VMEM((2, _H), jnp.float32),
        pltpu.VMEM_SHARED((_NPAD, _H), jnp.float32),
        pltpu.SemaphoreType.DMA,
        pltpu.SemaphoreType.DMA,
    ],
    compiler_params=pltpu.CompilerParams(use_tc_tiling_on_sc=False),
)(_sc_scatter_body)


def _tc_out_body(p, w2t, out):
  out[...] = jnp.dot(p[0] + p[1], w2t[...], preferred_element_type=jnp.float32)


_tc_out = pl.pallas_call(
    _tc_out_body,
    out_shape=jax.ShapeDtypeStruct((_N, _H), jnp.float32),
)


def kernel(x, edge_index, edge_weight, bn1_gamma, bn1_beta, W1,
           bn2_gamma, bn2_beta, W2):
  src = edge_index[0]
  dst = edge_index[1]
  xd, g, partials = _sc_gather(src, dst, x)
  z, ab2 = _tc_mlp1(
      partials,
      bn1_gamma.reshape(1, -1), bn1_beta.reshape(1, -1),
      W1[:, :_D].T, W1[:, _D:].T,
      bn2_gamma.reshape(1, -1), bn2_beta.reshape(1, -1),
      xd, g)
  p = _sc_scatter(dst, z, edge_weight, ab2)
  return _tc_out(p[:, :_N], W2.T)


# speedup vs baseline: 1.2519x; 1.0010x over previous
import functools

import jax
import jax.numpy as jnp
from jax import lax
from jax.experimental import pallas as pl
from jax.experimental.pallas import tpu as pltpu
from jax.experimental.pallas import tpu_sc as plsc

_N = 10000
_E = 320000
_D = 128
_H = 64
_EPS = 1e-5

_NC = 2
_NS = 16
_NW = _NC * _NS
_EPW = _E // _NW
_CH = 200
_NCHUNK = _EPW // _CH
_NSUPER = _NCHUNK // 2
_SUBS = (104, 96)

_SCH = 128
_NSCH = _EPW // _SCH
_STAIL = _EPW - _NSCH * _SCH
_NSS = _NSCH // 2
_NPAD = 10240
_RPT = _NPAD // _NS
_ZROWS = 128

_L = 16


def _leaky(t):
  return jnp.maximum(t, 0.2 * t)


def _sc_gather_body(src_hbm, dst_hbm, x_hbm, xd_out, g_out, stats_out,
                    idx_s0, idx_d0, idx_s1, idx_d1,
                    rows_s0, rows_d0, rows_s1, rows_d1,
                    acc, sem_g0, sem_g1, sem_w0, sem_w1):
  wid = lax.axis_index("s") * _NC + lax.axis_index("c")
  base0 = wid * _EPW

  zero = jnp.zeros((_L,), jnp.float32)
  for r in range(4):
    for g in range(_D // _L):
      acc[r, pl.ds(g * _L, _L)] = zero

  idx_s = (idx_s0, idx_s1)
  idx_d = (idx_d0, idx_d1)
  rows_s = (rows_s0, rows_s1)
  rows_d = (rows_d0, rows_d1)
  sem_g = (sem_g0, sem_g1)
  sem_w = (sem_w0, sem_w1)

  def fire_gathers(p, c):
    base = base0 + c * _CH
    pltpu.sync_copy(src_hbm.at[pl.ds(base, _CH)], idx_s[p])
    pltpu.sync_copy(dst_hbm.at[pl.ds(base, _CH)], idx_d[p])
    off = 0
    for sub in _SUBS:
      sl = pl.ds(off, sub)
      pltpu.async_copy(x_hbm.at[idx_s[p].at[sl]], rows_s[p].at[sl], sem_g[p])
      pltpu.async_copy(x_hbm.at[idx_d[p].at[sl]], rows_d[p].at[sl], sem_g[p])
      off += sub

  def drain_gathers(p):
    off = 0
    for sub in _SUBS:
      sl = pl.ds(off, sub)
      pltpu.make_async_copy(x_hbm.at[idx_s[p].at[sl]], rows_s[p].at[sl],
                            sem_g[p]).wait()
      pltpu.make_async_copy(x_hbm.at[idx_d[p].at[sl]], rows_d[p].at[sl],
                            sem_g[p]).wait()
      off += sub

  def fire_writes(p, c):
    base = base0 + c * _CH
    pltpu.async_copy(rows_d[p], xd_out.at[pl.ds(base, _CH)], sem_w[p])
    pltpu.async_copy(rows_s[p], g_out.at[pl.ds(base, _CH)], sem_w[p])

  def drain_writes(p):
    pltpu.make_async_copy(rows_d[p], xd_out.at[pl.ds(base0, _CH)],
                          sem_w[p]).wait()
    pltpu.make_async_copy(rows_s[p], g_out.at[pl.ds(base0, _CH)],
                          sem_w[p]).wait()

  _Q = _CH // 4

  def compute(p):
    rs = rows_s[p]
    rd = rows_d[p]
    for g in range(_D // _L):
      sl = pl.ds(g * _L, _L)

      def row_body(j, carry, rs=rs, rd=rd, sl=sl):
        out = []
        for q in range(4):
          ssd, sqd, ssg, sqg = carry[q]
          xd = rd[j + q * _Q, sl]
          xs = rs[j + q * _Q, sl]
          gd = xs - xd
          rs[j + q * _Q, sl] = gd
          out.append((ssd + xd, sqd + xd * xd, ssg + gd, sqg + gd * gd))
        return tuple(out)

      parts = plsc.parallel_loop(
          0, _Q, carry=((zero, zero, zero, zero),) * 4, unroll=2)(row_body)
      ssd = (parts[0][0] + parts[1][0]) + (parts[2][0] + parts[3][0])
      sqd = (parts[0][1] + parts[1][1]) + (parts[2][1] + parts[3][1])
      ssg = (parts[0][2] + parts[1][2]) + (parts[2][2] + parts[3][2])
      sqg = (parts[0][3] + parts[1][3]) + (parts[2][3] + parts[3][3])
      acc[0, sl] = acc[0, sl] + ssd
      acc[1, sl] = acc[1, sl] + sqd
      acc[2, sl] = acc[2, sl] + ssg
      acc[3, sl] = acc[3, sl] + sqg

  fire_gathers(0, 0)

  def super_body(t, _):
    @pl.when(t > 0)
    def _():
      drain_writes(1)
    fire_gathers(1, 2 * t + 1)

    drain_gathers(0)
    compute(0)
    fire_writes(0, 2 * t)

    @pl.when(t < _NSUPER - 1)
    def _():
      drain_writes(0)
      fire_gathers(0, 2 * t + 2)

    drain_gathers(1)
    compute(1)
    fire_writes(1, 2 * t + 1)
    return 0

  lax.fori_loop(0, _NSUPER, super_body, 0)
  drain_writes(0)
  drain_writes(1)
  pltpu.sync_copy(acc, stats_out.at[wid])


_sc_gather = functools.partial(
    pl.kernel,
    out_type=[
        jax.ShapeDtypeStruct((_E, _D), jnp.float32),
        jax.ShapeDtypeStruct((_E, _D), jnp.float32),
        jax.ShapeDtypeStruct((_NW, 4, _D), jnp.float32),
    ],
    mesh=plsc.VectorSubcoreMesh(core_axis_name="c", subcore_axis_name="s"),
    scratch_types=[
        pltpu.VMEM((_CH,), jnp.int32),
        pltpu.VMEM((_CH,), jnp.int32),
        pltpu.VMEM((_CH,), jnp.int32),
        pltpu.VMEM((_CH,), jnp.int32),
        pltpu.VMEM((_CH, _D), jnp.float32),
        pltpu.VMEM((_CH, _D), jnp.float32),
        pltpu.VMEM((_CH, _D), jnp.float32),
        pltpu.VMEM((_CH, _D), jnp.float32),
        pltpu.VMEM((4, _D), jnp.float32),
        pltpu.SemaphoreType.DMA,
        pltpu.SemaphoreType.DMA,
        pltpu.SemaphoreType.DMA,
        pltpu.SemaphoreType.DMA,
    ],
)(_sc_gather_body)


_BB = 2560
_NSTEP = _E // _BB


def _tc_mlp1_body(partials, g1, b1, w1at, w1bt, g2, b2,
                  xd_blk, g_blk, z_out, ab2_out, acc_s, acc_q):
  i = pl.program_id(0)
  sums = jnp.sum(partials[...], axis=0)
  mean_a = sums[0:1, :] / _E
  var_a = sums[1:2, :] / _E - mean_a * mean_a
  a1a = g1[:, 0:_D] * lax.rsqrt(var_a + _EPS)
  b1a = b1[:, 0:_D] - mean_a * a1a
  mean_b = sums[2:3, :] / _E
  var_b = sums[3:4, :] / _E - mean_b * mean_b
  a1b = g1[:, _D:] * lax.rsqrt(var_b + _EPS)
  b1b = b1[:, _D:] - mean_b * a1b

  ya = _leaky(xd_blk[...] * a1a + b1a)
  yb = _leaky(g_blk[...] * a1b + b1b)
  z = (jnp.dot(ya, w1at[...], preferred_element_type=jnp.float32)
       + jnp.dot(yb, w1bt[...], preferred_element_type=jnp.float32))
  z_out[...] = z

  s_blk = jnp.sum(z, axis=0, keepdims=True)
  q_blk = jnp.sum(z * z, axis=0, keepdims=True)

  @pl.when(i == 0)
  def _():
    acc_s[...] = s_blk
    acc_q[...] = q_blk

  @pl.when(i > 0)
  def _():
    acc_s[...] = acc_s[...] + s_blk
    acc_q[...] = acc_q[...] + q_blk

  @pl.when(i == _NSTEP - 1)
  def _():
    mean2 = acc_s[...] / _E
    var2 = acc_q[...] / _E - mean2 * mean2
    a2 = g2[...] * lax.rsqrt(var2 + _EPS)
    b2v = b2[...] - mean2 * a2
    ab2_out[0:1, :] = a2
    ab2_out[1:2, :] = b2v


_tc_mlp1 = pl.pallas_call(
    _tc_mlp1_body,
    grid=(_NSTEP,),
    in_specs=[
        pl.BlockSpec((_NW, 4, _D), lambda i: (0, 0, 0)),
        pl.BlockSpec((1, 2 * _D), lambda i: (0, 0)),
        pl.BlockSpec((1, 2 * _D), lambda i: (0, 0)),
        pl.BlockSpec((_D, _H), lambda i: (0, 0)),
        pl.BlockSpec((_D, _H), lambda i: (0, 0)),
        pl.BlockSpec((1, _H), lambda i: (0, 0)),
        pl.BlockSpec((1, _H), lambda i: (0, 0)),
        pl.BlockSpec((_BB, _D), lambda i: (i, 0)),
        pl.BlockSpec((_BB, _D), lambda i: (i, 0)),
    ],
    out_specs=[
        pl.BlockSpec((_BB, _H), lambda i: (i, 0)),
        pl.BlockSpec((2, _H), lambda i: (0, 0)),
    ],
    out_shape=[
        jax.ShapeDtypeStruct((_E, _H), jnp.float32),
        jax.ShapeDtypeStruct((2, _H), jnp.float32),
    ],
    scratch_shapes=[
        pltpu.VMEM((1, _H), jnp.float32),
        pltpu.VMEM((1, _H), jnp.float32),
    ],
)


def _sc_scatter_body(dst_hbm, z_hbm, ew_hbm, ab2_hbm, out_hbm,
                     idx0, idx1, idxt, vr0, vr1, vrt, zb0, zb1, zbt,
                     ew0, ew1, ewt, ab2_v,
                     shared_acc, sem_l0, sem_l1):
  cid = lax.axis_index("c")
  sid = lax.axis_index("s")
  wid = sid * _NC + cid
  base0 = wid * _EPW

  zero = jnp.zeros((_L,), jnp.float32)

  pltpu.sync_copy(ab2_hbm, ab2_v)

  def zrow(j, _):
    for g in range(_H // _L):
      vr0[j, pl.ds(g * _L, _L)] = zero
    return 0

  lax.fori_loop(0, _ZROWS, zrow, 0)
  for k in range(_RPT // _ZROWS):
    pltpu.sync_copy(vr0, shared_acc.at[pl.ds(sid * _RPT + k * _ZROWS, _ZROWS)])
  plsc.subcore_barrier()

  a2g = [ab2_v[0, pl.ds(k * _L, _L)] for k in range(_H // _L)]
  b2g = [ab2_v[1, pl.ds(k * _L, _L)] for k in range(_H // _L)]

  idx = (idx0, idx1)
  vr = (vr0, vr1)
  zb = (zb0, zb1)
  ew = (ew0, ew1)
  sem_l = (sem_l0, sem_l1)

  def fire_load(p, c):
    base = base0 + c * _SCH
    pltpu.async_copy(dst_hbm.at[pl.ds(base, _SCH)], idx[p].at[0], sem_l[p])
    pltpu.async_copy(z_hbm.at[pl.ds(base // 2, _SCH // 2)], zb[p], sem_l[p])
    pltpu.async_copy(ew_hbm.at[pl.ds(base, _SCH)], ew[p], sem_l[p])

  def drain_load(p):
    pltpu.make_async_copy(dst_hbm.at[pl.ds(base0, _SCH)], idx[p].at[0],
                          sem_l[p]).wait()
    pltpu.make_async_copy(z_hbm.at[pl.ds(base0, _SCH // 2)], zb[p],
                          sem_l[p]).wait()
    pltpu.make_async_copy(ew_hbm.at[pl.ds(base0, _SCH)], ew[p],
                          sem_l[p]).wait()

  def compute(p):
    buf = vr[p]
    src = zb[p]
    ewb = ew[p]

    def grp_body(gg):
      ewv = ewb[pl.ds(gg * _L, _L)]
      for i in range(_L):
        j = gg * _L + i
        ewj = ewv[i]
        srow = gg * (_L // 2) + i // 2
        scol = (i % 2) * _H
        for k in range(_H // _L):
          t = src[srow, pl.ds(scol + k * _L, _L)] * a2g[k] + b2g[k]
          buf[j, pl.ds(k * _L, _L)] = jnp.maximum(t, 0.2 * t) * ewj

    plsc.parallel_loop(0, _SCH // _L, unroll=1)(grp_body)

  def scatter(p):
    pltpu.sync_copy(vr[p], shared_acc.at[idx[p].at[0]], add=True)
    plsc.subcore_barrier()

  fire_load(0, 0)

  def super_body(t, _):
    fire_load(1, 2 * t + 1)
    drain_load(0)
    compute(0)
    scatter(0)

    @pl.when(t < _NSS - 1)
    def _():
      fire_load(0, 2 * t + 2)

    drain_load(1)
    compute(1)
    scatter(1)
    return 0

  lax.fori_loop(0, _NSS, super_body, 0)

  tbase = base0 + _NSCH * _SCH
  pltpu.sync_copy(dst_hbm.at[pl.ds(tbase, _STAIL)], idxt.at[0])
  pltpu.sync_copy(z_hbm.at[pl.ds(tbase // 2, _STAIL // 2)], zbt)
  pltpu.sync_copy(ew_hbm.at[pl.ds(tbase, _STAIL)], ewt)

  ewtv = ewt[pl.ds(0, _L)]
  for i in range(_STAIL):
    ewj = ewtv[i]
    for k in range(_H // _L):
      t = zbt[i // 2, pl.ds((i % 2) * _H + k * _L, _L)] * a2g[k] + b2g[k]
      vrt[i, pl.ds(k * _L, _L)] = jnp.maximum(t, 0.2 * t) * ewj
  pltpu.sync_copy(vrt, shared_acc.at[idxt.at[0]], add=True)
  plsc.subcore_barrier()

  pltpu.sync_copy(shared_acc.at[pl.ds(sid * _RPT, _RPT)],
                  out_hbm.at[cid, pl.ds(sid * _RPT, _RPT)])


_sc_scatter = functools.partial(
    pl.kernel,
    out_type=jax.ShapeDtypeStruct((_NC, _NPAD, _H), jnp.float32),
    mesh=plsc.VectorSubcoreMesh(core_axis_name="c", subcore_axis_name="s"),
    scratch_types=[
        pltpu.VMEM((1, _SCH), jnp.int32),
        pltpu.VMEM((1, _SCH), jnp.int32),
        pltpu.VMEM((1, _STAIL), jnp.int32),
        pltpu.VMEM((_SCH, _H), jnp.float32),
        pltpu.VMEM((_SCH, _H), jnp.float32),
        pltpu.VMEM((_STAIL, _H), jnp.float32),
        pltpu.VMEM((_SCH // 2, 2 * _H), jnp.float32),
        pltpu.VMEM((_SCH // 2, 2 * _H), jnp.float32),
        pltpu.VMEM((_STAIL // 2, 2 * _H), jnp.float32),
        pltpu.VMEM((_SCH,), jnp.float32),
        pltpu.VMEM((_SCH,), jnp.float32),
        pltpu.VMEM((_STAIL,), jnp.float32),
        pltpu.VMEM((2, _H), jnp.float32),
        pltpu.VMEM_SHARED((_NPAD, _H), jnp.float32),
        pltpu.SemaphoreType.DMA,
        pltpu.SemaphoreType.DMA,
    ],
    compiler_params=pltpu.CompilerParams(use_tc_tiling_on_sc=False),
)(_sc_scatter_body)


def _tc_out_body(p, w2t, out):
  out[...] = jnp.dot(p[0] + p[1], w2t[...], preferred_element_type=jnp.float32)


_tc_out = pl.pallas_call(
    _tc_out_body,
    out_shape=jax.ShapeDtypeStruct((_N, _H), jnp.float32),
)


def kernel(x, edge_index, edge_weight, bn1_gamma, bn1_beta, W1,
           bn2_gamma, bn2_beta, W2):
  src = edge_index[0]
  dst = edge_index[1]
  xd, g, partials = _sc_gather(src, dst, x)
  z, ab2 = _tc_mlp1(
      partials,
      bn1_gamma.reshape(1, -1), bn1_beta.reshape(1, -1),
      W1[:, :_D].T, W1[:, _D:].T,
      bn2_gamma.reshape(1, -1), bn2_beta.reshape(1, -1),
      xd, g)
  p = _sc_scatter(dst, z.reshape(_E // 2, 2 * _H), edge_weight, ab2)
  return _tc_out(p[:, :_N], W2.T)
